# baseline (speedup 1.0000x reference)
import functools

import jax
import jax.numpy as jnp
from jax import lax
from jax.experimental import pallas as pl
from jax.experimental.pallas import tpu as pltpu
from jax.experimental.pallas import tpu_sc as plsc

N = 10000
E = 320000
H = 128
NP = 10240
EP = 327680
EPW = EP // 32
CHUNK = 128
NCH = EPW // CHUNK
K0R = 158
K1R = 2
K0G = 220
K1G = 100
RPT = NP // 16
ZCH = RPT // CHUNK
EPS = 1e-5

_mesh = plsc.VectorSubcoreMesh(core_axis_name="c", subcore_axis_name="s")
_sc_params = pltpu.CompilerParams(needs_layout_passes=False,
                                  use_tc_tiling_on_sc=False)


def _f32(shape):
    return jax.ShapeDtypeStruct(shape, jnp.float32)


def _sc_cnt_body(dst_hbm, out_hbm, didx, ones_v, zrow, acc):
    c = lax.axis_index("c")
    s = lax.axis_index("s")
    w = c * 16 + s

    one16 = jnp.full((16,), 1.0, jnp.float32)
    z16 = jnp.zeros((16,), jnp.float32)

    @pl.loop(0, CHUNK)
    def _init(i):
        ones_v[i, :] = one16
        zrow[i, :] = z16

    @pl.loop(0, ZCH)
    def _zero(j):
        pltpu.sync_copy(zrow, acc.at[pl.ds(s * RPT + j * CHUNK, CHUNK)])

    plsc.subcore_barrier()

    @pl.loop(0, NCH)
    def _edges(i):
        off = w * EPW + i * CHUNK
        pltpu.sync_copy(dst_hbm.at[pl.ds(off, CHUNK)], didx)
        pltpu.sync_copy(ones_v, acc.at[didx], add=True)

    plsc.subcore_barrier()

    @pl.loop(0, ZCH)
    def _out(j):
        r = s * RPT + j * CHUNK
        pltpu.sync_copy(acc.at[pl.ds(r, CHUNK)], out_hbm.at[c, pl.ds(r, CHUNK)])


_sc_cnt = pl.kernel(
    _sc_cnt_body,
    out_type=_f32((2, NP, 16)),
    mesh=_mesh,
    compiler_params=_sc_params,
    scratch_types=[
        pltpu.VMEM((CHUNK,), jnp.int32),
        pltpu.VMEM((CHUNK, 16), jnp.float32),
        pltpu.VMEM((CHUNK, 16), jnp.float32),
        pltpu.VMEM_SHARED((NP, 16), jnp.float32),
    ],
)


def _sc_rows_body(src_hbm, dst_hbm, feat_hbm, out_hbm, sidx, didx, rows,
                  sem0, sem1, acc):
    c = lax.axis_index("c")
    s = lax.axis_index("s")
    w = c * 16 + s

    z16 = jnp.zeros((16,), jnp.float32)
    sems = (sem0, sem1)

    @pl.loop(0, CHUNK)
    def _init(i):
        for j in range(8):
            rows[0, i, pl.ds(j * 16, 16)] = z16

    @pl.loop(0, ZCH)
    def _zero(j):
        pltpu.sync_copy(rows.at[0],
                        acc.at[pl.ds(s * RPT + j * CHUNK, CHUNK)])

    plsc.subcore_barrier()

    base = jnp.where(c == 0, s * K0R, 16 * K0R + s * K1R)
    nch = jnp.where(c == 0, K0R, K1R)

    def _issue(b, ci):
        off = (base + ci) * CHUNK
        pltpu.sync_copy(src_hbm.at[pl.ds(off, CHUNK)], sidx.at[b])
        pltpu.sync_copy(dst_hbm.at[pl.ds(off, CHUNK)], didx.at[b])
        pltpu.async_copy(feat_hbm.at[sidx.at[b]], rows.at[b], sems[b])

    for b in range(2):
        _issue(b, b)

    @pl.loop(0, nch - 2, step=2)
    def _edges(i):
        for b in range(2):
            pltpu.make_async_copy(feat_hbm.at[sidx.at[b]], rows.at[b],
                                  sems[b]).wait()
            pltpu.sync_copy(rows.at[b], acc.at[didx.at[b]], add=True)
            _issue(b, i + b + 2)

    for b in range(2):
        pltpu.make_async_copy(feat_hbm.at[sidx.at[b]], rows.at[b],
                              sems[b]).wait()
        pltpu.sync_copy(rows.at[b], acc.at[didx.at[b]], add=True)

    plsc.subcore_barrier()

    @pl.loop(0, ZCH)
    def _out(j):
        r = s * RPT + j * CHUNK
        pltpu.sync_copy(acc.at[pl.ds(r, CHUNK)], out_hbm.at[c, pl.ds(r, CHUNK)])


_sc_rows = pl.kernel(
    _sc_rows_body,
    out_type=_f32((2, NP, H)),
    mesh=_mesh,
    compiler_params=_sc_params,
    scratch_types=[
        pltpu.VMEM((2, CHUNK), jnp.int32),
        pltpu.VMEM((2, CHUNK), jnp.int32),
        pltpu.VMEM((2, CHUNK, H), jnp.float32),
        pltpu.SemaphoreType.DMA,
        pltpu.SemaphoreType.DMA,
        pltpu.VMEM_SHARED((NP, H), jnp.float32),
    ],
)


CG = 64
NCHG = EPW // CG
ZCHG = RPT // CG


def _sc_gat_body(src_hbm, dst_hbm, xg_hbm, dl_hbm, ms_hbm, out_hbm,
                 sidx, didx, srows, dlg, msv, sem0, sem1, acc):
    c = lax.axis_index("c")
    s = lax.axis_index("s")
    w = c * 16 + s

    z16 = jnp.zeros((16,), jnp.float32)
    c0 = jnp.zeros((16,), jnp.int32)
    c1 = jnp.full((16,), 1, jnp.int32)
    c128 = jnp.full((16,), 128, jnp.int32)
    c136 = jnp.full((16,), 136, jnp.int32)
    iota16 = lax.iota(jnp.int32, 16)
    sems = (sem0, sem1)

    @pl.loop(0, CG)
    def _init(i):
        for j in range(9):
            srows[0, i, pl.ds(j * 16, 16)] = z16

    @pl.loop(0, ZCHG)
    def _zero(j):
        pltpu.sync_copy(srows.at[0], acc.at[pl.ds(s * RPT + j * CG, CG)])

    pltpu.sync_copy(ms_hbm, msv)
    plsc.subcore_barrier()

    base = jnp.where(c == 0, s * K0G, 16 * K0G + s * K1G)
    nchg = jnp.where(c == 0, K0G, K1G)

    def _issue(b, ci):
        off = (base + ci) * CG
        pltpu.sync_copy(src_hbm.at[pl.ds(off, CG)], sidx.at[b])
        pltpu.sync_copy(dst_hbm.at[pl.ds(off, CG)], didx.at[b])
        pltpu.async_copy(xg_hbm.at[sidx.at[b]], srows.at[b], sems[b])
        pltpu.async_copy(dl_hbm.at[didx.at[b]], dlg.at[b], sems[b])

    def _wait(b):
        pltpu.make_async_copy(xg_hbm.at[sidx.at[b]], srows.at[b],
                              sems[b]).wait()
        pltpu.make_async_copy(dl_hbm.at[didx.at[b]], dlg.at[b],
                              sems[b]).wait()

    def _compute(b):
        m0v = msv[0, pl.ds(0, 16)]
        m1v = msv[1, pl.ds(0, 16)]
        sb = srows.at[b]
        db = dlg.at[b]
        for g in range(CG // 16):
            rowi = g * 16 + iota16
            as0 = plsc.load_gather(sb, [rowi, c128])
            as1 = plsc.load_gather(sb, [rowi, c136])
            ad0 = plsc.load_gather(db, [rowi, c0])
            ad1 = plsc.load_gather(db, [rowi, c1])
            a0 = as0 + ad0
            a0 = jnp.where(a0 > 0.0, a0, a0 * 0.2)
            a1 = as1 + ad1
            a1 = jnp.where(a1 > 0.0, a1, a1 * 0.2)
            w0 = jnp.exp(a0 - m0v)
            w1 = jnp.exp(a1 - m1v)
            plsc.store_scatter(sb, [rowi, c128], w0)
            plsc.store_scatter(sb, [rowi, c136], w1)
            for e in range(16):
                r = g * 16 + e
                w0s = w0[e]
                w1s = w1[e]
                for j in range(8):
                    ws = w0s if j < 4 else w1s
                    sb[r, pl.ds(j * 16, 16)] = sb[r, pl.ds(j * 16, 16)] * ws
        pltpu.sync_copy(sb, acc.at[didx.at[b]], add=True)

    for b in range(2):
        _issue(b, b)

    @pl.loop(0, nchg - 2, step=2)
    def _edges(i):
        for b in range(2):
            _wait(b)
            _compute(b)
            _issue(b, i + b + 2)

    for b in range(2):
        _wait(b)
        _compute(b)

    plsc.subcore_barrier()

    @pl.loop(0, ZCHG)
    def _out(j):
        r = s * RPT + j * CG
        pltpu.sync_copy(acc.at[pl.ds(r, CG)], out_hbm.at[c, pl.ds(r, CG)])


_sc_gat = pl.kernel(
    _sc_gat_body,
    out_type=_f32((2, NP, 144)),
    mesh=_mesh,
    compiler_params=_sc_params,
    scratch_types=[
        pltpu.VMEM((2, CG), jnp.int32),
        pltpu.VMEM((2, CG), jnp.int32),
        pltpu.VMEM((2, CG, 144), jnp.float32),
        pltpu.VMEM((2, CG, 16), jnp.float32),
        pltpu.VMEM((8, 128), jnp.float32),
        pltpu.SemaphoreType.DMA,
        pltpu.SemaphoreType.DMA,
        pltpu.VMEM_SHARED((NP, 144), jnp.float32),
    ],
)


def _elu(v):
    return jnp.where(v > 0.0, v, jnp.exp(v) - 1.0)


def _bn(v, g, b, m, var):
    return (v - m) * lax.rsqrt(var + EPS) * g + b


def _tc1_body(x_ref, win_ref, bin_ref, wgcn_ref, h0_ref, xwg_ref):
    h0 = _elu(jnp.dot(x_ref[...], win_ref[...],
                      preferred_element_type=jnp.float32) + bin_ref[...])
    h0_ref[...] = h0
    xwg_ref[...] = jnp.dot(h0, wgcn_ref[...],
                           preferred_element_type=jnp.float32)


def _tc2_body(xwg_ref, cacc_ref, xws_ref):
    cnt = cacc_ref[0, :, 0:1] + cacc_ref[1, :, 0:1]
    dinv = lax.rsqrt(cnt + 1.0)
    xws_ref[...] = xwg_ref[...] * dinv


def _tc3_body(gacc_ref, cacc_ref, xwg_ref, wgat_ref, asrc_ref, adst_ref,
              bgcn_ref, g1_ref, b1_ref, m1_ref, v1_ref,
              h1_ref, xg_ref, dl_ref, ms_ref):
    cnt = cacc_ref[0, :, 0:1] + cacc_ref[1, :, 0:1]
    dinv = lax.rsqrt(cnt + 1.0)
    agg = gacc_ref[0] + gacc_ref[1]
    g = dinv * agg + dinv * dinv * xwg_ref[...] + bgcn_ref[...]
    h1 = _elu(_bn(g, g1_ref[...], b1_ref[...], m1_ref[...], v1_ref[...]))
    h1_ref[...] = h1
    xw2 = jnp.dot(h1, wgat_ref[...], preferred_element_type=jnp.float32)
    als0 = jnp.sum(xw2[:, :64] * asrc_ref[0:1, :], axis=-1, keepdims=True)
    als1 = jnp.sum(xw2[:, 64:] * asrc_ref[1:2, :], axis=-1, keepdims=True)
    ald0 = jnp.sum(xw2[:, :64] * adst_ref[0:1, :], axis=-1, keepdims=True)
    ald1 = jnp.sum(xw2[:, 64:] * adst_ref[1:2, :], axis=-1, keepdims=True)
    z7 = jnp.zeros((NP, 7), jnp.float32)
    xg_ref[...] = jnp.concatenate([xw2, als0, z7, als1, z7], axis=1)
    dl_ref[...] = jnp.concatenate(
        [ald0, ald1, jnp.zeros((NP, 14), jnp.float32)], axis=1)
    m0 = jnp.max(als0) + jnp.max(ald0)
    m1 = jnp.max(als1) + jnp.max(ald1)
    m0 = jnp.where(m0 > 0.0, m0, m0 * 0.2)
    m1 = jnp.where(m1 > 0.0, m1, m1 * 0.2)
    ms_ref[...] = jnp.concatenate(
        [jnp.full((1, 128), m0, jnp.float32),
         jnp.full((1, 128), m1, jnp.float32),
         jnp.zeros((6, 128), jnp.float32)], axis=0)


_TCB = 2560


def _tc4_body(aacc_ref, xg_ref, dl_ref, ms_ref, bgat_ref,
              g2_ref, b2_ref, m2_ref, v2_ref, h2_ref):
    ssum = aacc_ref[0] + aacc_ref[1]
    m0 = ms_ref[0, 0]
    m1 = ms_ref[1, 0]
    als = jnp.concatenate([xg_ref[:, 128:129], xg_ref[:, 136:137]], axis=1)
    a_self = als + dl_ref[:, 0:2]
    a_self = jnp.where(a_self > 0.0, a_self, a_self * 0.2)
    wself = jnp.exp(a_self - jnp.concatenate(
        [jnp.full((_TCB, 1), m0, jnp.float32),
         jnp.full((_TCB, 1), m1, jnp.float32)], axis=1))
    num0 = ssum[:, 0:64] + wself[:, 0:1] * xg_ref[:, 0:64]
    num1 = ssum[:, 64:128] + wself[:, 1:2] * xg_ref[:, 64:128]
    den0 = ssum[:, 128:129] + wself[:, 0:1] + 1e-16
    den1 = ssum[:, 136:137] + wself[:, 1:2] + 1e-16
    g = jnp.concatenate([num0 / den0, num1 / den1], axis=1) + bgat_ref[...]
    h2_ref[...] = _elu(_bn(g, g2_ref[...], b2_ref[...], m2_ref[...],
                           v2_ref[...]))


def _tc5_body(sacc_ref, cacc_ref, h2_ref, h0_ref, wsl_ref, wsr_ref, bs_ref,
              g3_ref, b3_ref, m3_ref, v3_ref, wres_ref, bres_ref,
              wc1_ref, bc1_ref, wc2_ref, bc2_ref, out_ref):
    cnt = cacc_ref[0, :, 0:1] + cacc_ref[1, :, 0:1]
    mean = (sacc_ref[0] + sacc_ref[1]) / jnp.maximum(cnt, 1.0)
    h3 = jnp.dot(mean, wsl_ref[...], preferred_element_type=jnp.float32) \
        + jnp.dot(h2_ref[...], wsr_ref[...],
                  preferred_element_type=jnp.float32) + bs_ref[...]
    h3 = _bn(h3, g3_ref[...], b3_ref[...], m3_ref[...], v3_ref[...])
    h4 = _elu(h3 + jnp.dot(h0_ref[...], wres_ref[...],
                           preferred_element_type=jnp.float32) + bres_ref[...])
    h5 = _elu(jnp.dot(h4, wc1_ref[...],
                      preferred_element_type=jnp.float32) + bc1_ref[...])
    out_ref[...] = jnp.dot(h5, wc2_ref[...],
                           preferred_element_type=jnp.float32) + bc2_ref[...]


def _tc_call(body, out_shapes):
    return pl.pallas_call(body, out_shape=out_shapes)


def kernel(x, edge_index, w_in, b_in, w_gcn, b_gcn, bn1_g, bn1_b, bn1_m,
           bn1_v, w_gat, a_src, a_dst, b_gat, bn2_g, bn2_b, bn2_m, bn2_v,
           w_sl, w_sr, b_sage, bn3_g, bn3_b, bn3_m, bn3_v, w_res, b_res,
           w_c1, b_c1, w_c2, b_c2):
    xp = jnp.zeros((NP, H), jnp.float32).at[:N].set(x)
    pad = jnp.full((EP - E,), N, jnp.int32)
    srcp = jnp.concatenate([edge_index[0], pad])
    dstp = jnp.concatenate([edge_index[1], pad])
    wc2p = jnp.zeros((32, 128), jnp.float32).at[:, :2].set(w_c2)
    bc2p = jnp.zeros((128,), jnp.float32).at[:2].set(b_c2)

    h0, xwg = _tc_call(_tc1_body, [_f32((NP, H)), _f32((NP, H))])(
        xp, w_in, b_in, w_gcn)

    cacc = _sc_cnt(dstp)

    xws = _tc_call(_tc2_body, _f32((NP, H)))(xwg, cacc)

    gacc = _sc_rows(srcp, dstp, xws)

    h1, xg, dl, ms = _tc_call(
        _tc3_body,
        [_f32((NP, H)), _f32((NP, 144)), _f32((NP, 16)), _f32((8, 128))])(
        gacc, cacc, xwg, w_gat, a_src, a_dst, b_gcn,
        bn1_g, bn1_b, bn1_m, bn1_v)

    aacc = _sc_gat(srcp, dstp, xg, dl, ms)

    h2 = pl.pallas_call(
        _tc4_body,
        grid=(NP // _TCB,),
        in_specs=[
            pl.BlockSpec((2, _TCB, 144), lambda i: (0, i, 0)),
            pl.BlockSpec((_TCB, 144), lambda i: (i, 0)),
            pl.BlockSpec((_TCB, 16), lambda i: (i, 0)),
            pl.BlockSpec((8, 128), lambda i: (0, 0)),
            pl.BlockSpec((H,), lambda i: (0,)),
            pl.BlockSpec((H,), lambda i: (0,)),
            pl.BlockSpec((H,), lambda i: (0,)),
            pl.BlockSpec((H,), lambda i: (0,)),
            pl.BlockSpec((H,), lambda i: (0,)),
        ],
        out_specs=pl.BlockSpec((_TCB, H), lambda i: (i, 0)),
        out_shape=_f32((NP, H)),
    )(aacc, xg, dl, ms, b_gat, bn2_g, bn2_b, bn2_m, bn2_v)

    sacc = _sc_rows(srcp, dstp, h2)

    out = _tc_call(_tc5_body, _f32((NP, 128)))(
        sacc, cacc, h2, h0, w_sl, w_sr, b_sage, bn3_g, bn3_b, bn3_m, bn3_v,
        w_res, b_res, w_c1, b_c1, wc2p, bc2p)

    return out[:N, :2]


# speedup vs baseline: 1.1164x; 1.1164x over previous
import functools

import jax
import jax.numpy as jnp
from jax import lax
from jax.experimental import pallas as pl
from jax.experimental.pallas import tpu as pltpu
from jax.experimental.pallas import tpu_sc as plsc

N = 10000
E = 320000
H = 128
NP = 10240
EP = 327680
EPW = EP // 32
CHUNK = 128
NCH = EPW // CHUNK
K0R = 136
K1R = 24
K0G = 220
K1G = 100
RPT = NP // 16
ZCH = RPT // CHUNK
EPS = 1e-5

_mesh = plsc.VectorSubcoreMesh(core_axis_name="c", subcore_axis_name="s")
_sc_params = pltpu.CompilerParams(needs_layout_passes=False,
                                  use_tc_tiling_on_sc=False)


def _f32(shape):
    return jax.ShapeDtypeStruct(shape, jnp.float32)


def _sc_cnt_body(dst_hbm, out_hbm, didx, ones_v, zrow, acc):
    c = lax.axis_index("c")
    s = lax.axis_index("s")
    w = c * 16 + s

    one16 = jnp.full((16,), 1.0, jnp.float32)
    z16 = jnp.zeros((16,), jnp.float32)

    @pl.loop(0, CHUNK)
    def _init(i):
        ones_v[i, :] = one16
        zrow[i, :] = z16

    @pl.loop(0, ZCH)
    def _zero(j):
        pltpu.sync_copy(zrow, acc.at[pl.ds(s * RPT + j * CHUNK, CHUNK)])

    plsc.subcore_barrier()

    @pl.loop(0, NCH)
    def _edges(i):
        off = w * EPW + i * CHUNK
        pltpu.sync_copy(dst_hbm.at[pl.ds(off, CHUNK)], didx)
        pltpu.sync_copy(ones_v, acc.at[didx], add=True)

    plsc.subcore_barrier()

    @pl.loop(0, ZCH)
    def _out(j):
        r = s * RPT + j * CHUNK
        pltpu.sync_copy(acc.at[pl.ds(r, CHUNK)], out_hbm.at[c, pl.ds(r, CHUNK)])


_sc_cnt = pl.kernel(
    _sc_cnt_body,
    out_type=_f32((2, NP, 16)),
    mesh=_mesh,
    compiler_params=_sc_params,
    scratch_types=[
        pltpu.VMEM((CHUNK,), jnp.int32),
        pltpu.VMEM((CHUNK, 16), jnp.float32),
        pltpu.VMEM((CHUNK, 16), jnp.float32),
        pltpu.VMEM_SHARED((NP, 16), jnp.float32),
    ],
)


def _sc_rows_body(src_hbm, dst_hbm, feat_hbm, out_hbm, sidx, didx, rows,
                  sem0, sem1, acc):
    c = lax.axis_index("c")
    s = lax.axis_index("s")
    w = c * 16 + s

    z16 = jnp.zeros((16,), jnp.float32)
    sems = (sem0, sem1)

    @pl.loop(0, CHUNK)
    def _init(i):
        for j in range(8):
            rows[0, i, pl.ds(j * 16, 16)] = z16

    @pl.loop(0, ZCH)
    def _zero(j):
        pltpu.sync_copy(rows.at[0],
                        acc.at[pl.ds(s * RPT + j * CHUNK, CHUNK)])

    plsc.subcore_barrier()

    base = jnp.where(c == 0, s * K0R, 16 * K0R + s * K1R)
    nch = jnp.where(c == 0, K0R, K1R)

    def _issue(b, ci):
        off = (base + ci) * CHUNK
        pltpu.sync_copy(src_hbm.at[pl.ds(off, CHUNK)], sidx.at[b])
        pltpu.sync_copy(dst_hbm.at[pl.ds(off, CHUNK)], didx.at[b])
        pltpu.async_copy(feat_hbm.at[sidx.at[b]], rows.at[b], sems[b])

    for b in range(2):
        _issue(b, b)

    @pl.loop(0, nch - 2, step=2)
    def _edges(i):
        for b in range(2):
            pltpu.make_async_copy(feat_hbm.at[sidx.at[b]], rows.at[b],
                                  sems[b]).wait()
            pltpu.sync_copy(rows.at[b], acc.at[didx.at[b]], add=True)
            _issue(b, i + b + 2)

    for b in range(2):
        pltpu.make_async_copy(feat_hbm.at[sidx.at[b]], rows.at[b],
                              sems[b]).wait()
        pltpu.sync_copy(rows.at[b], acc.at[didx.at[b]], add=True)

    plsc.subcore_barrier()

    @pl.loop(0, ZCH)
    def _out(j):
        r = s * RPT + j * CHUNK
        pltpu.sync_copy(acc.at[pl.ds(r, CHUNK)], out_hbm.at[c, pl.ds(r, CHUNK)])


_sc_rows = pl.kernel(
    _sc_rows_body,
    out_type=_f32((2, NP, H)),
    mesh=_mesh,
    compiler_params=_sc_params,
    scratch_types=[
        pltpu.VMEM((2, CHUNK), jnp.int32),
        pltpu.VMEM((2, CHUNK), jnp.int32),
        pltpu.VMEM((2, CHUNK, H), jnp.float32),
        pltpu.SemaphoreType.DMA,
        pltpu.SemaphoreType.DMA,
        pltpu.VMEM_SHARED((NP, H), jnp.float32),
    ],
)


CG = 64
NCHG = EPW // CG
ZCHG = RPT // CG


def _sc_gat_body(src_hbm, dst_hbm, xg_hbm, dl_hbm, ms_hbm, out_hbm,
                 sidx, didx, srows, dlg, msv, sem0, sem1, acc):
    c = lax.axis_index("c")
    s = lax.axis_index("s")
    w = c * 16 + s

    z16 = jnp.zeros((16,), jnp.float32)
    c0 = jnp.zeros((16,), jnp.int32)
    c1 = jnp.full((16,), 1, jnp.int32)
    c128 = jnp.full((16,), 128, jnp.int32)
    c136 = jnp.full((16,), 136, jnp.int32)
    iota16 = lax.iota(jnp.int32, 16)
    sems = (sem0, sem1)

    @pl.loop(0, CG)
    def _init(i):
        for j in range(9):
            srows[0, i, pl.ds(j * 16, 16)] = z16

    @pl.loop(0, ZCHG)
    def _zero(j):
        pltpu.sync_copy(srows.at[0], acc.at[pl.ds(s * RPT + j * CG, CG)])

    pltpu.sync_copy(ms_hbm, msv)
    plsc.subcore_barrier()

    base = jnp.where(c == 0, s * K0G, 16 * K0G + s * K1G)
    nchg = jnp.where(c == 0, K0G, K1G)

    def _issue(b, ci):
        off = (base + ci) * CG
        pltpu.sync_copy(src_hbm.at[pl.ds(off, CG)], sidx.at[b])
        pltpu.sync_copy(dst_hbm.at[pl.ds(off, CG)], didx.at[b])
        pltpu.async_copy(xg_hbm.at[sidx.at[b]], srows.at[b], sems[b])
        pltpu.async_copy(dl_hbm.at[didx.at[b]], dlg.at[b], sems[b])

    def _wait(b):
        pltpu.make_async_copy(xg_hbm.at[sidx.at[b]], srows.at[b],
                              sems[b]).wait()
        pltpu.make_async_copy(dl_hbm.at[didx.at[b]], dlg.at[b],
                              sems[b]).wait()

    def _compute(b):
        m0v = msv[0, pl.ds(0, 16)]
        m1v = msv[1, pl.ds(0, 16)]
        sb = srows.at[b]
        db = dlg.at[b]
        for g in range(CG // 16):
            rowi = g * 16 + iota16
            as0 = plsc.load_gather(sb, [rowi, c128])
            as1 = plsc.load_gather(sb, [rowi, c136])
            ad0 = plsc.load_gather(db, [rowi, c0])
            ad1 = plsc.load_gather(db, [rowi, c1])
            a0 = as0 + ad0
            a0 = jnp.where(a0 > 0.0, a0, a0 * 0.2)
            a1 = as1 + ad1
            a1 = jnp.where(a1 > 0.0, a1, a1 * 0.2)
            w0 = jnp.exp(a0 - m0v)
            w1 = jnp.exp(a1 - m1v)
            plsc.store_scatter(sb, [rowi, c128], w0)
            plsc.store_scatter(sb, [rowi, c136], w1)
            for e in range(16):
                r = g * 16 + e
                w0s = w0[e]
                w1s = w1[e]
                for j in range(8):
                    ws = w0s if j < 4 else w1s
                    sb[r, pl.ds(j * 16, 16)] = sb[r, pl.ds(j * 16, 16)] * ws
        pltpu.sync_copy(sb, acc.at[didx.at[b]], add=True)

    for b in range(2):
        _issue(b, b)

    @pl.loop(0, nchg - 2, step=2)
    def _edges(i):
        for b in range(2):
            _wait(b)
            _compute(b)
            _issue(b, i + b + 2)

    for b in range(2):
        _wait(b)
        _compute(b)

    plsc.subcore_barrier()

    @pl.loop(0, ZCHG)
    def _out(j):
        r = s * RPT + j * CG
        pltpu.sync_copy(acc.at[pl.ds(r, CG)], out_hbm.at[c, pl.ds(r, CG)])


_sc_gat = pl.kernel(
    _sc_gat_body,
    out_type=_f32((2, NP, 144)),
    mesh=_mesh,
    compiler_params=_sc_params,
    scratch_types=[
        pltpu.VMEM((2, CG), jnp.int32),
        pltpu.VMEM((2, CG), jnp.int32),
        pltpu.VMEM((2, CG, 144), jnp.float32),
        pltpu.VMEM((2, CG, 16), jnp.float32),
        pltpu.VMEM((8, 128), jnp.float32),
        pltpu.SemaphoreType.DMA,
        pltpu.SemaphoreType.DMA,
        pltpu.VMEM_SHARED((NP, 144), jnp.float32),
    ],
)


def _elu(v):
    return jnp.where(v > 0.0, v, jnp.exp(v) - 1.0)


def _bn(v, g, b, m, var):
    return (v - m) * lax.rsqrt(var + EPS) * g + b


def _tc1_body(x_ref, win_ref, bin_ref, wgcn_ref, h0_ref, xwg_ref):
    h0 = _elu(jnp.dot(x_ref[...], win_ref[...],
                      preferred_element_type=jnp.float32) + bin_ref[...])
    h0_ref[...] = h0
    xwg_ref[...] = jnp.dot(h0, wgcn_ref[...],
                           preferred_element_type=jnp.float32)


def _tc2_body(xwg_ref, cacc_ref, xws_ref):
    cnt = cacc_ref[0, :, 0:1] + cacc_ref[1, :, 0:1]
    dinv = lax.rsqrt(cnt + 1.0)
    xws_ref[...] = xwg_ref[...] * dinv


def _tc3_body(gacc_ref, cacc_ref, xwg_ref, wgat_ref, asrc_ref, adst_ref,
              bgcn_ref, g1_ref, b1_ref, m1_ref, v1_ref,
              h1_ref, xg_ref, dl_ref, ms_ref):
    cnt = cacc_ref[0, :, 0:1] + cacc_ref[1, :, 0:1]
    dinv = lax.rsqrt(cnt + 1.0)
    agg = gacc_ref[0] + gacc_ref[1]
    g = dinv * agg + dinv * dinv * xwg_ref[...] + bgcn_ref[...]
    h1 = _elu(_bn(g, g1_ref[...], b1_ref[...], m1_ref[...], v1_ref[...]))
    h1_ref[...] = h1
    xw2 = jnp.dot(h1, wgat_ref[...], preferred_element_type=jnp.float32)
    als0 = jnp.sum(xw2[:, :64] * asrc_ref[0:1, :], axis=-1, keepdims=True)
    als1 = jnp.sum(xw2[:, 64:] * asrc_ref[1:2, :], axis=-1, keepdims=True)
    ald0 = jnp.sum(xw2[:, :64] * adst_ref[0:1, :], axis=-1, keepdims=True)
    ald1 = jnp.sum(xw2[:, 64:] * adst_ref[1:2, :], axis=-1, keepdims=True)
    z7 = jnp.zeros((NP, 7), jnp.float32)
    xg_ref[...] = jnp.concatenate([xw2, als0, z7, als1, z7], axis=1)
    dl_ref[...] = jnp.concatenate(
        [ald0, ald1, jnp.zeros((NP, 14), jnp.float32)], axis=1)
    m0 = jnp.max(als0) + jnp.max(ald0)
    m1 = jnp.max(als1) + jnp.max(ald1)
    m0 = jnp.where(m0 > 0.0, m0, m0 * 0.2)
    m1 = jnp.where(m1 > 0.0, m1, m1 * 0.2)
    ms_ref[...] = jnp.concatenate(
        [jnp.full((1, 128), m0, jnp.float32),
         jnp.full((1, 128), m1, jnp.float32),
         jnp.zeros((6, 128), jnp.float32)], axis=0)


_TCB = 2560


def _tc4_body(aacc_ref, xg_ref, dl_ref, ms_ref, bgat_ref,
              g2_ref, b2_ref, m2_ref, v2_ref, h2_ref):
    ssum = aacc_ref[0] + aacc_ref[1]
    m0 = ms_ref[0, 0]
    m1 = ms_ref[1, 0]
    als = jnp.concatenate([xg_ref[:, 128:129], xg_ref[:, 136:137]], axis=1)
    a_self = als + dl_ref[:, 0:2]
    a_self = jnp.where(a_self > 0.0, a_self, a_self * 0.2)
    wself = jnp.exp(a_self - jnp.concatenate(
        [jnp.full((_TCB, 1), m0, jnp.float32),
         jnp.full((_TCB, 1), m1, jnp.float32)], axis=1))
    num0 = ssum[:, 0:64] + wself[:, 0:1] * xg_ref[:, 0:64]
    num1 = ssum[:, 64:128] + wself[:, 1:2] * xg_ref[:, 64:128]
    den0 = ssum[:, 128:129] + wself[:, 0:1] + 1e-16
    den1 = ssum[:, 136:137] + wself[:, 1:2] + 1e-16
    g = jnp.concatenate([num0 / den0, num1 / den1], axis=1) + bgat_ref[...]
    h2_ref[...] = _elu(_bn(g, g2_ref[...], b2_ref[...], m2_ref[...],
                           v2_ref[...]))


def _tc5_body(sacc_ref, cacc_ref, h2_ref, h0_ref, wsl_ref, wsr_ref, bs_ref,
              g3_ref, b3_ref, m3_ref, v3_ref, wres_ref, bres_ref,
              wc1_ref, bc1_ref, wc2_ref, bc2_ref, out_ref):
    cnt = cacc_ref[0, :, 0:1] + cacc_ref[1, :, 0:1]
    mean = (sacc_ref[0] + sacc_ref[1]) / jnp.maximum(cnt, 1.0)
    h3 = jnp.dot(mean, wsl_ref[...], preferred_element_type=jnp.float32) \
        + jnp.dot(h2_ref[...], wsr_ref[...],
                  preferred_element_type=jnp.float32) + bs_ref[...]
    h3 = _bn(h3, g3_ref[...], b3_ref[...], m3_ref[...], v3_ref[...])
    h4 = _elu(h3 + jnp.dot(h0_ref[...], wres_ref[...],
                           preferred_element_type=jnp.float32) + bres_ref[...])
    h5 = _elu(jnp.dot(h4, wc1_ref[...],
                      preferred_element_type=jnp.float32) + bc1_ref[...])
    out_ref[...] = jnp.dot(h5, wc2_ref[...],
                           preferred_element_type=jnp.float32) + bc2_ref[...]


def _tc_call(body, out_shapes):
    return pl.pallas_call(body, out_shape=out_shapes)


def kernel(x, edge_index, w_in, b_in, w_gcn, b_gcn, bn1_g, bn1_b, bn1_m,
           bn1_v, w_gat, a_src, a_dst, b_gat, bn2_g, bn2_b, bn2_m, bn2_v,
           w_sl, w_sr, b_sage, bn3_g, bn3_b, bn3_m, bn3_v, w_res, b_res,
           w_c1, b_c1, w_c2, b_c2):
    xp = jnp.zeros((NP, H), jnp.float32).at[:N].set(x)
    pad = jnp.full((EP - E,), N, jnp.int32)
    srcp = jnp.concatenate([edge_index[0], pad])
    dstp = jnp.concatenate([edge_index[1], pad])
    wc2p = jnp.zeros((32, 128), jnp.float32).at[:, :2].set(w_c2)
    bc2p = jnp.zeros((128,), jnp.float32).at[:2].set(b_c2)

    h0, xwg = _tc_call(_tc1_body, [_f32((NP, H)), _f32((NP, H))])(
        xp, w_in, b_in, w_gcn)

    cacc = _sc_cnt(dstp)

    xws = _tc_call(_tc2_body, _f32((NP, H)))(xwg, cacc)

    gacc = _sc_rows(srcp, dstp, xws)

    h1, xg, dl, ms = _tc_call(
        _tc3_body,
        [_f32((NP, H)), _f32((NP, 144)), _f32((NP, 16)), _f32((8, 128))])(
        gacc, cacc, xwg, w_gat, a_src, a_dst, b_gcn,
        bn1_g, bn1_b, bn1_m, bn1_v)

    aacc = _sc_gat(srcp, dstp, xg, dl, ms)

    h2 = pl.pallas_call(
        _tc4_body,
        grid=(NP // _TCB,),
        in_specs=[
            pl.BlockSpec((2, _TCB, 144), lambda i: (0, i, 0)),
            pl.BlockSpec((_TCB, 144), lambda i: (i, 0)),
            pl.BlockSpec((_TCB, 16), lambda i: (i, 0)),
            pl.BlockSpec((8, 128), lambda i: (0, 0)),
            pl.BlockSpec((H,), lambda i: (0,)),
            pl.BlockSpec((H,), lambda i: (0,)),
            pl.BlockSpec((H,), lambda i: (0,)),
            pl.BlockSpec((H,), lambda i: (0,)),
            pl.BlockSpec((H,), lambda i: (0,)),
        ],
        out_specs=pl.BlockSpec((_TCB, H), lambda i: (i, 0)),
        out_shape=_f32((NP, H)),
    )(aacc, xg, dl, ms, b_gat, bn2_g, bn2_b, bn2_m, bn2_v)

    sacc = _sc_rows(srcp, dstp, h2)

    out = _tc_call(_tc5_body, _f32((NP, 128)))(
        sacc, cacc, h2, h0, w_sl, w_sr, b_sage, bn3_g, bn3_b, bn3_m, bn3_v,
        w_res, b_res, w_c1, b_c1, wc2p, bc2p)

    return out[:N, :2]


# speedup vs baseline: 1.2290x; 1.1008x over previous
import functools

import jax
import jax.numpy as jnp
from jax import lax
from jax.experimental import pallas as pl
from jax.experimental.pallas import tpu as pltpu
from jax.experimental.pallas import tpu_sc as plsc

N = 10000
E = 320000
H = 128
NP = 10240
EP = 327680
EPW = EP // 32
CHUNK = 128
NCH = EPW // CHUNK
K0R = 144
K1R = 16
K0G = 220
K1G = 100
RPT = NP // 16
ZCH = RPT // CHUNK
EPS = 1e-5

_mesh = plsc.VectorSubcoreMesh(core_axis_name="c", subcore_axis_name="s")
_sc_params = pltpu.CompilerParams(needs_layout_passes=False,
                                  use_tc_tiling_on_sc=False)


def _f32(shape):
    return jax.ShapeDtypeStruct(shape, jnp.float32)


def _sc_cnt_body(dst_hbm, out_hbm, didx, ones_v, zrow, acc):
    c = lax.axis_index("c")
    s = lax.axis_index("s")
    w = c * 16 + s

    one16 = jnp.full((16,), 1.0, jnp.float32)
    z16 = jnp.zeros((16,), jnp.float32)

    @pl.loop(0, CHUNK)
    def _init(i):
        ones_v[i, :] = one16
        zrow[i, :] = z16

    @pl.loop(0, ZCH)
    def _zero(j):
        pltpu.sync_copy(zrow, acc.at[pl.ds(s * RPT + j * CHUNK, CHUNK)])

    plsc.subcore_barrier()

    @pl.loop(0, NCH)
    def _edges(i):
        off = w * EPW + i * CHUNK
        pltpu.sync_copy(dst_hbm.at[pl.ds(off, CHUNK)], didx)
        pltpu.sync_copy(ones_v, acc.at[didx], add=True)

    plsc.subcore_barrier()

    @pl.loop(0, ZCH)
    def _out(j):
        r = s * RPT + j * CHUNK
        pltpu.sync_copy(acc.at[pl.ds(r, CHUNK)], out_hbm.at[c, pl.ds(r, CHUNK)])


_sc_cnt = pl.kernel(
    _sc_cnt_body,
    out_type=_f32((2, NP, 16)),
    mesh=_mesh,
    compiler_params=_sc_params,
    scratch_types=[
        pltpu.VMEM((CHUNK,), jnp.int32),
        pltpu.VMEM((CHUNK, 16), jnp.float32),
        pltpu.VMEM((CHUNK, 16), jnp.float32),
        pltpu.VMEM_SHARED((NP, 16), jnp.float32),
    ],
)


def _sc_rows_body(src_hbm, dst_hbm, feat_hbm, out_hbm, sidx, didx, rows,
                  sem0, sem1, acc):
    c = lax.axis_index("c")
    s = lax.axis_index("s")
    w = c * 16 + s

    z16 = jnp.zeros((16,), jnp.float32)
    sems = (sem0, sem1)

    @pl.loop(0, CHUNK)
    def _init(i):
        for j in range(8):
            rows[0, i, pl.ds(j * 16, 16)] = z16

    @pl.loop(0, ZCH)
    def _zero(j):
        pltpu.sync_copy(rows.at[0],
                        acc.at[pl.ds(s * RPT + j * CHUNK, CHUNK)])

    plsc.subcore_barrier()

    base = jnp.where(c == 0, s * K0R, 16 * K0R + s * K1R)
    nch = jnp.where(c == 0, K0R, K1R)

    def _issue(b, ci):
        off = (base + ci) * CHUNK
        pltpu.sync_copy(src_hbm.at[pl.ds(off, CHUNK)], sidx.at[b])
        pltpu.sync_copy(dst_hbm.at[pl.ds(off, CHUNK)], didx.at[b])
        pltpu.async_copy(feat_hbm.at[sidx.at[b]], rows.at[b], sems[b])

    for b in range(2):
        _issue(b, b)

    @pl.loop(0, nch - 2, step=2)
    def _edges(i):
        for b in range(2):
            pltpu.make_async_copy(feat_hbm.at[sidx.at[b]], rows.at[b],
                                  sems[b]).wait()
            pltpu.sync_copy(rows.at[b], acc.at[didx.at[b]], add=True)
            _issue(b, i + b + 2)

    for b in range(2):
        pltpu.make_async_copy(feat_hbm.at[sidx.at[b]], rows.at[b],
                              sems[b]).wait()
        pltpu.sync_copy(rows.at[b], acc.at[didx.at[b]], add=True)

    plsc.subcore_barrier()

    @pl.loop(0, ZCH)
    def _out(j):
        r = s * RPT + j * CHUNK
        pltpu.sync_copy(acc.at[pl.ds(r, CHUNK)], out_hbm.at[c, pl.ds(r, CHUNK)])


_sc_rows = pl.kernel(
    _sc_rows_body,
    out_type=_f32((2, NP, H)),
    mesh=_mesh,
    compiler_params=_sc_params,
    scratch_types=[
        pltpu.VMEM((2, CHUNK), jnp.int32),
        pltpu.VMEM((2, CHUNK), jnp.int32),
        pltpu.VMEM((2, CHUNK, H), jnp.float32),
        pltpu.SemaphoreType.DMA,
        pltpu.SemaphoreType.DMA,
        pltpu.VMEM_SHARED((NP, H), jnp.float32),
    ],
)


CG = 64
NCHG = EPW // CG
ZCHG = RPT // CG


def _sc_gat_body(src_hbm, dst_hbm, xg_hbm, dl_hbm, ms_hbm, out_hbm,
                 sidx, didx, srows, dlg, msv, sem0, sem1, acc):
    c = lax.axis_index("c")
    s = lax.axis_index("s")
    w = c * 16 + s

    z16 = jnp.zeros((16,), jnp.float32)
    c0 = jnp.zeros((16,), jnp.int32)
    c1 = jnp.full((16,), 1, jnp.int32)
    c128 = jnp.full((16,), 128, jnp.int32)
    c136 = jnp.full((16,), 136, jnp.int32)
    iota16 = lax.iota(jnp.int32, 16)
    sems = (sem0, sem1)

    @pl.loop(0, CG)
    def _init(i):
        for j in range(9):
            srows[0, i, pl.ds(j * 16, 16)] = z16

    @pl.loop(0, ZCHG)
    def _zero(j):
        pltpu.sync_copy(srows.at[0], acc.at[pl.ds(s * RPT + j * CG, CG)])

    pltpu.sync_copy(ms_hbm, msv)
    plsc.subcore_barrier()

    base = jnp.where(c == 0, s * K0G, 16 * K0G + s * K1G)
    nchg = jnp.where(c == 0, K0G, K1G)

    def _issue(b, ci):
        off = (base + ci) * CG
        pltpu.sync_copy(src_hbm.at[pl.ds(off, CG)], sidx.at[b])
        pltpu.sync_copy(dst_hbm.at[pl.ds(off, CG)], didx.at[b])
        pltpu.async_copy(xg_hbm.at[sidx.at[b]], srows.at[b], sems[b])
        pltpu.async_copy(dl_hbm.at[didx.at[b]], dlg.at[b], sems[b])

    def _wait(b):
        pltpu.make_async_copy(xg_hbm.at[sidx.at[b]], srows.at[b],
                              sems[b]).wait()
        pltpu.make_async_copy(dl_hbm.at[didx.at[b]], dlg.at[b],
                              sems[b]).wait()

    def _compute(b):
        m0v = msv[0, pl.ds(0, 16)]
        m1v = msv[1, pl.ds(0, 16)]
        sb = srows.at[b]
        db = dlg.at[b]
        for g in range(CG // 16):
            rowi = g * 16 + iota16
            as0 = plsc.load_gather(sb, [rowi, c128])
            as1 = plsc.load_gather(sb, [rowi, c136])
            ad0 = plsc.load_gather(db, [rowi, c0])
            ad1 = plsc.load_gather(db, [rowi, c1])
            a0 = as0 + ad0
            a0 = jnp.where(a0 > 0.0, a0, a0 * 0.2)
            a1 = as1 + ad1
            a1 = jnp.where(a1 > 0.0, a1, a1 * 0.2)
            w0 = jnp.exp(a0 - m0v)
            w1 = jnp.exp(a1 - m1v)
            plsc.store_scatter(sb, [rowi, c128], w0)
            plsc.store_scatter(sb, [rowi, c136], w1)
            for e in range(16):
                r = g * 16 + e
                w0s = w0[e]
                w1s = w1[e]
                for j in range(8):
                    ws = w0s if j < 4 else w1s
                    sb[r, pl.ds(j * 16, 16)] = sb[r, pl.ds(j * 16, 16)] * ws
        pltpu.sync_copy(sb, acc.at[didx.at[b]], add=True)

    for b in range(2):
        _issue(b, b)

    @pl.loop(0, nchg - 2, step=2)
    def _edges(i):
        for b in range(2):
            _wait(b)
            _compute(b)
            _issue(b, i + b + 2)

    for b in range(2):
        _wait(b)
        _compute(b)

    plsc.subcore_barrier()

    @pl.loop(0, ZCHG)
    def _out(j):
        r = s * RPT + j * CG
        pltpu.sync_copy(acc.at[pl.ds(r, CG)], out_hbm.at[c, pl.ds(r, CG)])


_sc_gat = pl.kernel(
    _sc_gat_body,
    out_type=_f32((2, NP, 144)),
    mesh=_mesh,
    compiler_params=_sc_params,
    scratch_types=[
        pltpu.VMEM((2, CG), jnp.int32),
        pltpu.VMEM((2, CG), jnp.int32),
        pltpu.VMEM((2, CG, 144), jnp.float32),
        pltpu.VMEM((2, CG, 16), jnp.float32),
        pltpu.VMEM((8, 128), jnp.float32),
        pltpu.SemaphoreType.DMA,
        pltpu.SemaphoreType.DMA,
        pltpu.VMEM_SHARED((NP, 144), jnp.float32),
    ],
)


def _elu(v):
    return jnp.where(v > 0.0, v, jnp.exp(v) - 1.0)


def _bn(v, g, b, m, var):
    return (v - m) * lax.rsqrt(var + EPS) * g + b


def _tc1_body(x_ref, win_ref, bin_ref, wgcn_ref, h0_ref, xwg_ref):
    h0 = _elu(jnp.dot(x_ref[...], win_ref[...],
                      preferred_element_type=jnp.float32) + bin_ref[...])
    h0_ref[...] = h0
    xwg_ref[...] = jnp.dot(h0, wgcn_ref[...],
                           preferred_element_type=jnp.float32)


def _tc2_body(xwg_ref, cacc_ref, xws_ref):
    cnt = cacc_ref[0, :, 0:1] + cacc_ref[1, :, 0:1]
    dinv = lax.rsqrt(cnt + 1.0)
    xws_ref[...] = xwg_ref[...] * dinv


def _tc3_body(gacc_ref, cacc_ref, xwg_ref, wgat_ref, asrc_ref, adst_ref,
              bgcn_ref, g1_ref, b1_ref, m1_ref, v1_ref,
              h1_ref, xg_ref, dl_ref, ms_ref):
    cnt = cacc_ref[0, :, 0:1] + cacc_ref[1, :, 0:1]
    dinv = lax.rsqrt(cnt + 1.0)
    agg = gacc_ref[0] + gacc_ref[1]
    g = dinv * agg + dinv * dinv * xwg_ref[...] + bgcn_ref[...]
    h1 = _elu(_bn(g, g1_ref[...], b1_ref[...], m1_ref[...], v1_ref[...]))
    h1_ref[...] = h1
    xw2 = jnp.dot(h1, wgat_ref[...], preferred_element_type=jnp.float32)
    als0 = jnp.sum(xw2[:, :64] * asrc_ref[0:1, :], axis=-1, keepdims=True)
    als1 = jnp.sum(xw2[:, 64:] * asrc_ref[1:2, :], axis=-1, keepdims=True)
    ald0 = jnp.sum(xw2[:, :64] * adst_ref[0:1, :], axis=-1, keepdims=True)
    ald1 = jnp.sum(xw2[:, 64:] * adst_ref[1:2, :], axis=-1, keepdims=True)
    z7 = jnp.zeros((NP, 7), jnp.float32)
    xg_ref[...] = jnp.concatenate([xw2, als0, z7, als1, z7], axis=1)
    dl_ref[...] = jnp.concatenate(
        [ald0, ald1, jnp.zeros((NP, 14), jnp.float32)], axis=1)
    m0 = jnp.max(als0) + jnp.max(ald0)
    m1 = jnp.max(als1) + jnp.max(ald1)
    m0 = jnp.where(m0 > 0.0, m0, m0 * 0.2)
    m1 = jnp.where(m1 > 0.0, m1, m1 * 0.2)
    ms_ref[...] = jnp.concatenate(
        [jnp.full((1, 128), m0, jnp.float32),
         jnp.full((1, 128), m1, jnp.float32),
         jnp.zeros((6, 128), jnp.float32)], axis=0)


_TCB = 2560


def _tc4_body(aacc_ref, xg_ref, dl_ref, ms_ref, bgat_ref,
              g2_ref, b2_ref, m2_ref, v2_ref, h2_ref):
    ssum = aacc_ref[0] + aacc_ref[1]
    m0 = ms_ref[0, 0]
    m1 = ms_ref[1, 0]
    als = jnp.concatenate([xg_ref[:, 128:129], xg_ref[:, 136:137]], axis=1)
    a_self = als + dl_ref[:, 0:2]
    a_self = jnp.where(a_self > 0.0, a_self, a_self * 0.2)
    wself = jnp.exp(a_self - jnp.concatenate(
        [jnp.full((_TCB, 1), m0, jnp.float32),
         jnp.full((_TCB, 1), m1, jnp.float32)], axis=1))
    num0 = ssum[:, 0:64] + wself[:, 0:1] * xg_ref[:, 0:64]
    num1 = ssum[:, 64:128] + wself[:, 1:2] * xg_ref[:, 64:128]
    den0 = ssum[:, 128:129] + wself[:, 0:1] + 1e-16
    den1 = ssum[:, 136:137] + wself[:, 1:2] + 1e-16
    g = jnp.concatenate([num0 / den0, num1 / den1], axis=1) + bgat_ref[...]
    h2_ref[...] = _elu(_bn(g, g2_ref[...], b2_ref[...], m2_ref[...],
                           v2_ref[...]))


def _tc5_body(sacc_ref, cacc_ref, h2_ref, h0_ref, wsl_ref, wsr_ref, bs_ref,
              g3_ref, b3_ref, m3_ref, v3_ref, wres_ref, bres_ref,
              wc1_ref, bc1_ref, wc2_ref, bc2_ref, out_ref):
    cnt = cacc_ref[0, :, 0:1] + cacc_ref[1, :, 0:1]
    mean = (sacc_ref[0] + sacc_ref[1]) / jnp.maximum(cnt, 1.0)
    h3 = jnp.dot(mean, wsl_ref[...], preferred_element_type=jnp.float32) \
        + jnp.dot(h2_ref[...], wsr_ref[...],
                  preferred_element_type=jnp.float32) + bs_ref[...]
    h3 = _bn(h3, g3_ref[...], b3_ref[...], m3_ref[...], v3_ref[...])
    h4 = _elu(h3 + jnp.dot(h0_ref[...], wres_ref[...],
                           preferred_element_type=jnp.float32) + bres_ref[...])
    h5 = _elu(jnp.dot(h4, wc1_ref[...],
                      preferred_element_type=jnp.float32) + bc1_ref[...])
    out_ref[...] = jnp.dot(h5, wc2_ref[...],
                           preferred_element_type=jnp.float32) + bc2_ref[...]


def _tc_call(body, out_shapes):
    return pl.pallas_call(body, out_shape=out_shapes)


def kernel(x, edge_index, w_in, b_in, w_gcn, b_gcn, bn1_g, bn1_b, bn1_m,
           bn1_v, w_gat, a_src, a_dst, b_gat, bn2_g, bn2_b, bn2_m, bn2_v,
           w_sl, w_sr, b_sage, bn3_g, bn3_b, bn3_m, bn3_v, w_res, b_res,
           w_c1, b_c1, w_c2, b_c2):
    xp = jnp.zeros((NP, H), jnp.float32).at[:N].set(x)
    pad = jnp.full((EP - E,), N, jnp.int32)
    srcp = jnp.concatenate([edge_index[0], pad])
    dstp = jnp.concatenate([edge_index[1], pad])
    wc2p = jnp.zeros((32, 128), jnp.float32).at[:, :2].set(w_c2)
    bc2p = jnp.zeros((128,), jnp.float32).at[:2].set(b_c2)

    h0, xwg = _tc_call(_tc1_body, [_f32((NP, H)), _f32((NP, H))])(
        xp, w_in, b_in, w_gcn)

    cacc = _sc_cnt(dstp)

    xws = _tc_call(_tc2_body, _f32((NP, H)))(xwg, cacc)

    gacc = _sc_rows(srcp, dstp, xws)

    h1, xg, dl, ms = _tc_call(
        _tc3_body,
        [_f32((NP, H)), _f32((NP, 144)), _f32((NP, 16)), _f32((8, 128))])(
        gacc, cacc, xwg, w_gat, a_src, a_dst, b_gcn,
        bn1_g, bn1_b, bn1_m, bn1_v)

    aacc = _sc_gat(srcp, dstp, xg, dl, ms)

    h2 = pl.pallas_call(
        _tc4_body,
        grid=(NP // _TCB,),
        in_specs=[
            pl.BlockSpec((2, _TCB, 144), lambda i: (0, i, 0)),
            pl.BlockSpec((_TCB, 144), lambda i: (i, 0)),
            pl.BlockSpec((_TCB, 16), lambda i: (i, 0)),
            pl.BlockSpec((8, 128), lambda i: (0, 0)),
            pl.BlockSpec((H,), lambda i: (0,)),
            pl.BlockSpec((H,), lambda i: (0,)),
            pl.BlockSpec((H,), lambda i: (0,)),
            pl.BlockSpec((H,), lambda i: (0,)),
            pl.BlockSpec((H,), lambda i: (0,)),
        ],
        out_specs=pl.BlockSpec((_TCB, H), lambda i: (i, 0)),
        out_shape=_f32((NP, H)),
    )(aacc, xg, dl, ms, b_gat, bn2_g, bn2_b, bn2_m, bn2_v)

    sacc = _sc_rows(srcp, dstp, h2)

    out = _tc_call(_tc5_body, _f32((NP, 128)))(
        sacc, cacc, h2, h0, w_sl, w_sr, b_sage, bn3_g, bn3_b, bn3_m, bn3_v,
        w_res, b_res, w_c1, b_c1, wc2p, bc2p)

    return out[:N, :2]


# speedup vs baseline: 1.2509x; 1.0178x over previous
import functools

import jax
import jax.numpy as jnp
from jax import lax
from jax.experimental import pallas as pl
from jax.experimental.pallas import tpu as pltpu
from jax.experimental.pallas import tpu_sc as plsc

N = 10000
E = 320000
H = 128
NP = 10240
EP = 327680
EPW = EP // 32
CHUNK = 128
NCH = EPW // CHUNK
K0R = 150
K1R = 10
K0G = 220
K1G = 100
RPT = NP // 16
ZCH = RPT // CHUNK
EPS = 1e-5

_mesh = plsc.VectorSubcoreMesh(core_axis_name="c", subcore_axis_name="s")
_sc_params = pltpu.CompilerParams(needs_layout_passes=False,
                                  use_tc_tiling_on_sc=False)


def _f32(shape):
    return jax.ShapeDtypeStruct(shape, jnp.float32)


def _sc_cnt_body(dst_hbm, out_hbm, didx, ones_v, zrow, acc):
    c = lax.axis_index("c")
    s = lax.axis_index("s")
    w = c * 16 + s

    one16 = jnp.full((16,), 1.0, jnp.float32)
    z16 = jnp.zeros((16,), jnp.float32)

    @pl.loop(0, CHUNK)
    def _init(i):
        ones_v[i, :] = one16
        zrow[i, :] = z16

    @pl.loop(0, ZCH)
    def _zero(j):
        pltpu.sync_copy(zrow, acc.at[pl.ds(s * RPT + j * CHUNK, CHUNK)])

    plsc.subcore_barrier()

    @pl.loop(0, NCH)
    def _edges(i):
        off = w * EPW + i * CHUNK
        pltpu.sync_copy(dst_hbm.at[pl.ds(off, CHUNK)], didx)
        pltpu.sync_copy(ones_v, acc.at[didx], add=True)

    plsc.subcore_barrier()

    @pl.loop(0, ZCH)
    def _out(j):
        r = s * RPT + j * CHUNK
        pltpu.sync_copy(acc.at[pl.ds(r, CHUNK)], out_hbm.at[c, pl.ds(r, CHUNK)])


_sc_cnt = pl.kernel(
    _sc_cnt_body,
    out_type=_f32((2, NP, 16)),
    mesh=_mesh,
    compiler_params=_sc_params,
    scratch_types=[
        pltpu.VMEM((CHUNK,), jnp.int32),
        pltpu.VMEM((CHUNK, 16), jnp.float32),
        pltpu.VMEM((CHUNK, 16), jnp.float32),
        pltpu.VMEM_SHARED((NP, 16), jnp.float32),
    ],
)


def _sc_rows_body(src_hbm, dst_hbm, feat_hbm, out_hbm, sidx, didx, rows,
                  sem0, sem1, acc):
    c = lax.axis_index("c")
    s = lax.axis_index("s")
    w = c * 16 + s

    z16 = jnp.zeros((16,), jnp.float32)
    sems = (sem0, sem1)

    @pl.loop(0, CHUNK)
    def _init(i):
        for j in range(8):
            rows[0, i, pl.ds(j * 16, 16)] = z16

    @pl.loop(0, ZCH)
    def _zero(j):
        pltpu.sync_copy(rows.at[0],
                        acc.at[pl.ds(s * RPT + j * CHUNK, CHUNK)])

    plsc.subcore_barrier()

    base = jnp.where(c == 0, s * K0R, 16 * K0R + s * K1R)
    nch = jnp.where(c == 0, K0R, K1R)

    def _issue(b, ci):
        off = (base + ci) * CHUNK
        pltpu.sync_copy(src_hbm.at[pl.ds(off, CHUNK)], sidx.at[b])
        pltpu.sync_copy(dst_hbm.at[pl.ds(off, CHUNK)], didx.at[b])
        pltpu.async_copy(feat_hbm.at[sidx.at[b]], rows.at[b], sems[b])

    for b in range(2):
        _issue(b, b)

    @pl.loop(0, nch - 2, step=2)
    def _edges(i):
        for b in range(2):
            pltpu.make_async_copy(feat_hbm.at[sidx.at[b]], rows.at[b],
                                  sems[b]).wait()
            pltpu.sync_copy(rows.at[b], acc.at[didx.at[b]], add=True)
            _issue(b, i + b + 2)

    for b in range(2):
        pltpu.make_async_copy(feat_hbm.at[sidx.at[b]], rows.at[b],
                              sems[b]).wait()
        pltpu.sync_copy(rows.at[b], acc.at[didx.at[b]], add=True)

    plsc.subcore_barrier()

    @pl.loop(0, ZCH)
    def _out(j):
        r = s * RPT + j * CHUNK
        pltpu.sync_copy(acc.at[pl.ds(r, CHUNK)], out_hbm.at[c, pl.ds(r, CHUNK)])


_sc_rows = pl.kernel(
    _sc_rows_body,
    out_type=_f32((2, NP, H)),
    mesh=_mesh,
    compiler_params=_sc_params,
    scratch_types=[
        pltpu.VMEM((2, CHUNK), jnp.int32),
        pltpu.VMEM((2, CHUNK), jnp.int32),
        pltpu.VMEM((2, CHUNK, H), jnp.float32),
        pltpu.SemaphoreType.DMA,
        pltpu.SemaphoreType.DMA,
        pltpu.VMEM_SHARED((NP, H), jnp.float32),
    ],
)


CG = 64
NCHG = EPW // CG
ZCHG = RPT // CG


def _sc_gat_body(src_hbm, dst_hbm, xg_hbm, dl_hbm, ms_hbm, out_hbm,
                 sidx, didx, srows, dlg, msv, sem0, sem1, acc):
    c = lax.axis_index("c")
    s = lax.axis_index("s")
    w = c * 16 + s

    z16 = jnp.zeros((16,), jnp.float32)
    c0 = jnp.zeros((16,), jnp.int32)
    c1 = jnp.full((16,), 1, jnp.int32)
    c128 = jnp.full((16,), 128, jnp.int32)
    c136 = jnp.full((16,), 136, jnp.int32)
    iota16 = lax.iota(jnp.int32, 16)
    sems = (sem0, sem1)

    @pl.loop(0, CG)
    def _init(i):
        for j in range(9):
            srows[0, i, pl.ds(j * 16, 16)] = z16

    @pl.loop(0, ZCHG)
    def _zero(j):
        pltpu.sync_copy(srows.at[0], acc.at[pl.ds(s * RPT + j * CG, CG)])

    pltpu.sync_copy(ms_hbm, msv)
    plsc.subcore_barrier()

    base = jnp.where(c == 0, s * K0G, 16 * K0G + s * K1G)
    nchg = jnp.where(c == 0, K0G, K1G)

    def _issue(b, ci):
        off = (base + ci) * CG
        pltpu.sync_copy(src_hbm.at[pl.ds(off, CG)], sidx.at[b])
        pltpu.sync_copy(dst_hbm.at[pl.ds(off, CG)], didx.at[b])
        pltpu.async_copy(xg_hbm.at[sidx.at[b]], srows.at[b], sems[b])
        pltpu.async_copy(dl_hbm.at[didx.at[b]], dlg.at[b], sems[b])

    def _wait(b):
        pltpu.make_async_copy(xg_hbm.at[sidx.at[b]], srows.at[b],
                              sems[b]).wait()
        pltpu.make_async_copy(dl_hbm.at[didx.at[b]], dlg.at[b],
                              sems[b]).wait()

    def _compute(b):
        m0v = msv[0, pl.ds(0, 16)]
        m1v = msv[1, pl.ds(0, 16)]
        sb = srows.at[b]
        db = dlg.at[b]
        for g in range(CG // 16):
            rowi = g * 16 + iota16
            as0 = plsc.load_gather(sb, [rowi, c128])
            as1 = plsc.load_gather(sb, [rowi, c136])
            ad0 = plsc.load_gather(db, [rowi, c0])
            ad1 = plsc.load_gather(db, [rowi, c1])
            a0 = as0 + ad0
            a0 = jnp.where(a0 > 0.0, a0, a0 * 0.2)
            a1 = as1 + ad1
            a1 = jnp.where(a1 > 0.0, a1, a1 * 0.2)
            w0 = jnp.exp(a0 - m0v)
            w1 = jnp.exp(a1 - m1v)
            plsc.store_scatter(sb, [rowi, c128], w0)
            plsc.store_scatter(sb, [rowi, c136], w1)
            for e in range(16):
                r = g * 16 + e
                w0s = w0[e]
                w1s = w1[e]
                for j in range(8):
                    ws = w0s if j < 4 else w1s
                    sb[r, pl.ds(j * 16, 16)] = sb[r, pl.ds(j * 16, 16)] * ws
        pltpu.sync_copy(sb, acc.at[didx.at[b]], add=True)

    for b in range(2):
        _issue(b, b)

    @pl.loop(0, nchg - 2, step=2)
    def _edges(i):
        for b in range(2):
            _wait(b)
            _compute(b)
            _issue(b, i + b + 2)

    for b in range(2):
        _wait(b)
        _compute(b)

    plsc.subcore_barrier()

    @pl.loop(0, ZCHG)
    def _out(j):
        r = s * RPT + j * CG
        pltpu.sync_copy(acc.at[pl.ds(r, CG)], out_hbm.at[c, pl.ds(r, CG)])


_sc_gat = pl.kernel(
    _sc_gat_body,
    out_type=_f32((2, NP, 144)),
    mesh=_mesh,
    compiler_params=_sc_params,
    scratch_types=[
        pltpu.VMEM((2, CG), jnp.int32),
        pltpu.VMEM((2, CG), jnp.int32),
        pltpu.VMEM((2, CG, 144), jnp.float32),
        pltpu.VMEM((2, CG, 16), jnp.float32),
        pltpu.VMEM((8, 128), jnp.float32),
        pltpu.SemaphoreType.DMA,
        pltpu.SemaphoreType.DMA,
        pltpu.VMEM_SHARED((NP, 144), jnp.float32),
    ],
)


def _elu(v):
    return jnp.where(v > 0.0, v, jnp.exp(v) - 1.0)


def _bn(v, g, b, m, var):
    return (v - m) * lax.rsqrt(var + EPS) * g + b


def _tc1_body(x_ref, win_ref, bin_ref, wgcn_ref, h0_ref, xwg_ref):
    h0 = _elu(jnp.dot(x_ref[...], win_ref[...],
                      preferred_element_type=jnp.float32) + bin_ref[...])
    h0_ref[...] = h0
    xwg_ref[...] = jnp.dot(h0, wgcn_ref[...],
                           preferred_element_type=jnp.float32)


def _tc2_body(xwg_ref, cacc_ref, xws_ref):
    cnt = cacc_ref[0, :, 0:1] + cacc_ref[1, :, 0:1]
    dinv = lax.rsqrt(cnt + 1.0)
    xws_ref[...] = xwg_ref[...] * dinv


def _tc3_body(gacc_ref, cacc_ref, xwg_ref, wgat_ref, asrc_ref, adst_ref,
              bgcn_ref, g1_ref, b1_ref, m1_ref, v1_ref,
              h1_ref, xg_ref, dl_ref, ms_ref):
    cnt = cacc_ref[0, :, 0:1] + cacc_ref[1, :, 0:1]
    dinv = lax.rsqrt(cnt + 1.0)
    agg = gacc_ref[0] + gacc_ref[1]
    g = dinv * agg + dinv * dinv * xwg_ref[...] + bgcn_ref[...]
    h1 = _elu(_bn(g, g1_ref[...], b1_ref[...], m1_ref[...], v1_ref[...]))
    h1_ref[...] = h1
    xw2 = jnp.dot(h1, wgat_ref[...], preferred_element_type=jnp.float32)
    als0 = jnp.sum(xw2[:, :64] * asrc_ref[0:1, :], axis=-1, keepdims=True)
    als1 = jnp.sum(xw2[:, 64:] * asrc_ref[1:2, :], axis=-1, keepdims=True)
    ald0 = jnp.sum(xw2[:, :64] * adst_ref[0:1, :], axis=-1, keepdims=True)
    ald1 = jnp.sum(xw2[:, 64:] * adst_ref[1:2, :], axis=-1, keepdims=True)
    z7 = jnp.zeros((NP, 7), jnp.float32)
    xg_ref[...] = jnp.concatenate([xw2, als0, z7, als1, z7], axis=1)
    dl_ref[...] = jnp.concatenate(
        [ald0, ald1, jnp.zeros((NP, 14), jnp.float32)], axis=1)
    m0 = jnp.max(als0) + jnp.max(ald0)
    m1 = jnp.max(als1) + jnp.max(ald1)
    m0 = jnp.where(m0 > 0.0, m0, m0 * 0.2)
    m1 = jnp.where(m1 > 0.0, m1, m1 * 0.2)
    ms_ref[...] = jnp.concatenate(
        [jnp.full((1, 128), m0, jnp.float32),
         jnp.full((1, 128), m1, jnp.float32),
         jnp.zeros((6, 128), jnp.float32)], axis=0)


_TCB = 2560


def _tc4_body(aacc_ref, xg_ref, dl_ref, ms_ref, bgat_ref,
              g2_ref, b2_ref, m2_ref, v2_ref, h2_ref):
    ssum = aacc_ref[0] + aacc_ref[1]
    m0 = ms_ref[0, 0]
    m1 = ms_ref[1, 0]
    als = jnp.concatenate([xg_ref[:, 128:129], xg_ref[:, 136:137]], axis=1)
    a_self = als + dl_ref[:, 0:2]
    a_self = jnp.where(a_self > 0.0, a_self, a_self * 0.2)
    wself = jnp.exp(a_self - jnp.concatenate(
        [jnp.full((_TCB, 1), m0, jnp.float32),
         jnp.full((_TCB, 1), m1, jnp.float32)], axis=1))
    num0 = ssum[:, 0:64] + wself[:, 0:1] * xg_ref[:, 0:64]
    num1 = ssum[:, 64:128] + wself[:, 1:2] * xg_ref[:, 64:128]
    den0 = ssum[:, 128:129] + wself[:, 0:1] + 1e-16
    den1 = ssum[:, 136:137] + wself[:, 1:2] + 1e-16
    g = jnp.concatenate([num0 / den0, num1 / den1], axis=1) + bgat_ref[...]
    h2_ref[...] = _elu(_bn(g, g2_ref[...], b2_ref[...], m2_ref[...],
                           v2_ref[...]))


def _tc5_body(sacc_ref, cacc_ref, h2_ref, h0_ref, wsl_ref, wsr_ref, bs_ref,
              g3_ref, b3_ref, m3_ref, v3_ref, wres_ref, bres_ref,
              wc1_ref, bc1_ref, wc2_ref, bc2_ref, out_ref):
    cnt = cacc_ref[0, :, 0:1] + cacc_ref[1, :, 0:1]
    mean = (sacc_ref[0] + sacc_ref[1]) / jnp.maximum(cnt, 1.0)
    h3 = jnp.dot(mean, wsl_ref[...], preferred_element_type=jnp.float32) \
        + jnp.dot(h2_ref[...], wsr_ref[...],
                  preferred_element_type=jnp.float32) + bs_ref[...]
    h3 = _bn(h3, g3_ref[...], b3_ref[...], m3_ref[...], v3_ref[...])
    h4 = _elu(h3 + jnp.dot(h0_ref[...], wres_ref[...],
                           preferred_element_type=jnp.float32) + bres_ref[...])
    h5 = _elu(jnp.dot(h4, wc1_ref[...],
                      preferred_element_type=jnp.float32) + bc1_ref[...])
    out_ref[...] = jnp.dot(h5, wc2_ref[...],
                           preferred_element_type=jnp.float32) + bc2_ref[...]


def _tc_call(body, out_shapes):
    return pl.pallas_call(body, out_shape=out_shapes)


def kernel(x, edge_index, w_in, b_in, w_gcn, b_gcn, bn1_g, bn1_b, bn1_m,
           bn1_v, w_gat, a_src, a_dst, b_gat, bn2_g, bn2_b, bn2_m, bn2_v,
           w_sl, w_sr, b_sage, bn3_g, bn3_b, bn3_m, bn3_v, w_res, b_res,
           w_c1, b_c1, w_c2, b_c2):
    xp = jnp.zeros((NP, H), jnp.float32).at[:N].set(x)
    pad = jnp.full((EP - E,), N, jnp.int32)
    srcp = jnp.concatenate([edge_index[0], pad])
    dstp = jnp.concatenate([edge_index[1], pad])
    wc2p = jnp.zeros((32, 128), jnp.float32).at[:, :2].set(w_c2)
    bc2p = jnp.zeros((128,), jnp.float32).at[:2].set(b_c2)

    h0, xwg = _tc_call(_tc1_body, [_f32((NP, H)), _f32((NP, H))])(
        xp, w_in, b_in, w_gcn)

    cacc = _sc_cnt(dstp)

    xws = _tc_call(_tc2_body, _f32((NP, H)))(xwg, cacc)

    gacc = _sc_rows(srcp, dstp, xws)

    h1, xg, dl, ms = _tc_call(
        _tc3_body,
        [_f32((NP, H)), _f32((NP, 144)), _f32((NP, 16)), _f32((8, 128))])(
        gacc, cacc, xwg, w_gat, a_src, a_dst, b_gcn,
        bn1_g, bn1_b, bn1_m, bn1_v)

    aacc = _sc_gat(srcp, dstp, xg, dl, ms)

    h2 = pl.pallas_call(
        _tc4_body,
        grid=(NP // _TCB,),
        in_specs=[
            pl.BlockSpec((2, _TCB, 144), lambda i: (0, i, 0)),
            pl.BlockSpec((_TCB, 144), lambda i: (i, 0)),
            pl.BlockSpec((_TCB, 16), lambda i: (i, 0)),
            pl.BlockSpec((8, 128), lambda i: (0, 0)),
            pl.BlockSpec((H,), lambda i: (0,)),
            pl.BlockSpec((H,), lambda i: (0,)),
            pl.BlockSpec((H,), lambda i: (0,)),
            pl.BlockSpec((H,), lambda i: (0,)),
            pl.BlockSpec((H,), lambda i: (0,)),
        ],
        out_specs=pl.BlockSpec((_TCB, H), lambda i: (i, 0)),
        out_shape=_f32((NP, H)),
    )(aacc, xg, dl, ms, b_gat, bn2_g, bn2_b, bn2_m, bn2_v)

    sacc = _sc_rows(srcp, dstp, h2)

    out = _tc_call(_tc5_body, _f32((NP, 128)))(
        sacc, cacc, h2, h0, w_sl, w_sr, b_sage, bn3_g, bn3_b, bn3_m, bn3_v,
        w_res, b_res, w_c1, b_c1, wc2p, bc2p)

    return out[:N, :2]
